# Initial kernel scaffold; baseline (speedup 1.0000x reference)
import jax
import jax.numpy as jnp
from jax.experimental import pallas as pl


def kernel(pos_embedding_matrix, position_labels):
    raise NotImplementedError("write your pallas kernel here")


# speedup vs baseline: 3.9654x; 3.9654x over previous
import functools

import jax
import jax.numpy as jnp
from jax import lax
from jax.experimental import pallas as pl
from jax.experimental.pallas import tpu as pltpu
from jax.experimental.pallas import tpu_sc as plsc

_MAX_LENGTH = 2048
_HIDDEN = 64

_NC = 2
_NS = 16
_NW = _NC * _NS

_CHUNK = 512


def _gather_body(nchunks, idx_hbm, table_hbm, out_hbm, idx_v, rows_v, sem):
    wid = lax.axis_index("s") * _NC + lax.axis_index("c")
    per_w = nchunks * _CHUNK
    base = wid * per_w

    def step(i, carry):
        off = base + i * _CHUNK
        pltpu.sync_copy(idx_hbm.at[pl.ds(off, _CHUNK)], idx_v)
        pltpu.async_copy(table_hbm.at[idx_v], rows_v, sem).wait()
        pltpu.sync_copy(rows_v, out_hbm.at[pl.ds(off, _CHUNK)])
        return carry

    lax.fori_loop(0, nchunks, step, 0)


def kernel(pos_embedding_matrix, position_labels):
    b, s = position_labels.shape
    flat = position_labels.reshape(-1).astype(jnp.int32)
    n = flat.shape[0]
    assert n % (_NW * _CHUNK) == 0
    nchunks = n // (_NW * _CHUNK)

    mesh = plsc.VectorSubcoreMesh(core_axis_name="c", subcore_axis_name="s")
    run = pl.kernel(
        functools.partial(_gather_body, nchunks),
        mesh=mesh,
        compiler_params=pltpu.CompilerParams(use_tc_tiling_on_sc=False),
        out_type=jax.ShapeDtypeStruct((n, _HIDDEN), jnp.float32),
        scratch_types=[
            pltpu.VMEM((_CHUNK,), jnp.int32),
            pltpu.VMEM((_CHUNK, _HIDDEN), jnp.float32),
            pltpu.SemaphoreType.DMA,
        ],
    )
    out = run(flat, pos_embedding_matrix)
    return out.reshape(b, s, _HIDDEN)


# speedup vs baseline: 4.0159x; 1.0127x over previous
import functools

import jax
import jax.numpy as jnp
from jax import lax
from jax.experimental import pallas as pl
from jax.experimental.pallas import tpu as pltpu
from jax.experimental.pallas import tpu_sc as plsc

_HIDDEN = 64

_NC = 2
_NS = 16
_NW = _NC * _NS

_CHUNK = 512


def _gather_body(nchunks, idx_hbm, table_hbm, out_hbm,
                 idx_v, rows0, rows1, gsem0, gsem1, wsem0, wsem1):
    wid = lax.axis_index("s") * _NC + lax.axis_index("c")
    per_w = nchunks * _CHUNK
    base = wid * per_w

    rows = (rows0, rows1)
    gsem = (gsem0, gsem1)
    wsem = (wsem0, wsem1)

    pltpu.sync_copy(idx_hbm.at[pl.ds(base, per_w)], idx_v)

    def idx_slice(g):
        return idx_v.at[pl.ds(g * _CHUNK, _CHUNK)]

    def out_slice(g):
        return out_hbm.at[pl.ds(base + g * _CHUNK, _CHUNK)]

    pltpu.async_copy(table_hbm.at[idx_slice(0)], rows0, gsem0)

    def step(j, carry):
        for b in range(2):
            g = 2 * j + b
            nb = 1 - b

            @pl.when(g + 1 < nchunks)
            def _fire():
                @pl.when(g >= 1)
                def _drain():
                    pltpu.make_async_copy(
                        rows[nb], out_slice(g), wsem[nb]).wait()
                pltpu.async_copy(
                    table_hbm.at[idx_slice(g + 1)], rows[nb], gsem[nb])

            pltpu.make_async_copy(
                table_hbm.at[idx_slice(g)], rows[b], gsem[b]).wait()
            pltpu.async_copy(rows[b], out_slice(g), wsem[b])
        return carry

    lax.fori_loop(0, nchunks // 2, step, 0)

    pltpu.make_async_copy(rows0, out_slice(nchunks - 2), wsem0).wait()
    pltpu.make_async_copy(rows1, out_slice(nchunks - 1), wsem1).wait()


def kernel(pos_embedding_matrix, position_labels):
    b, s = position_labels.shape
    flat = position_labels.reshape(-1).astype(jnp.int32)
    n = flat.shape[0]
    assert n % (_NW * 2 * _CHUNK) == 0
    nchunks = n // (_NW * _CHUNK)

    mesh = plsc.VectorSubcoreMesh(core_axis_name="c", subcore_axis_name="s")
    run = pl.kernel(
        functools.partial(_gather_body, nchunks),
        mesh=mesh,
        compiler_params=pltpu.CompilerParams(use_tc_tiling_on_sc=False),
        out_type=jax.ShapeDtypeStruct((n, _HIDDEN), jnp.float32),
        scratch_types=[
            pltpu.VMEM((n // _NW,), jnp.int32),
            pltpu.VMEM((_CHUNK, _HIDDEN), jnp.float32),
            pltpu.VMEM((_CHUNK, _HIDDEN), jnp.float32),
            pltpu.SemaphoreType.DMA,
            pltpu.SemaphoreType.DMA,
            pltpu.SemaphoreType.DMA,
            pltpu.SemaphoreType.DMA,
        ],
    )
    out = run(flat, pos_embedding_matrix)
    return out.reshape(b, s, _HIDDEN)
